# baseline (speedup 1.0000x reference)
import functools

import jax
import jax.numpy as jnp
from jax import lax
from jax.experimental import pallas as pl
from jax.experimental.pallas import tpu as pltpu
from jax.experimental.pallas import tpu_sc as plsc

_F0 = 1000000
_B = 16384
_D = 32
_L = 16


@functools.cache
def _make_sc_kernel(num_cores, num_subcores):
    nw = num_cores * num_subcores
    bpw = _B // nw
    mesh = plsc.VectorSubcoreMesh(core_axis_name="c", subcore_axis_name="s")

    @functools.partial(
        pl.kernel,
        mesh=mesh,
        out_type=jax.ShapeDtypeStruct((_B,), jnp.float32),
        compiler_params=pltpu.CompilerParams(
            needs_layout_passes=False, use_tc_tiling_on_sc=False),
        scratch_types=[
            pltpu.VMEM((bpw,), jnp.int32),
            pltpu.VMEM((bpw,), jnp.int32),
            pltpu.VMEM((bpw,), jnp.int32),
            pltpu.VMEM((bpw, _D), jnp.float32),
            pltpu.VMEM((bpw, _D), jnp.float32),
            pltpu.VMEM((bpw,), jnp.float32),
            pltpu.VMEM((bpw,), jnp.float32),
            pltpu.VMEM((_D,), jnp.float32),
            pltpu.VMEM((_L,), jnp.float32),
            pltpu.VMEM((bpw + _L,), jnp.float32),
            pltpu.SemaphoreType.DMA,
            pltpu.SemaphoreType.DMA,
            pltpu.SemaphoreType.DMA,
            pltpu.SemaphoreType.DMA,
        ],
    )
    def sc_kernel(uidx_hbm, iidx_hbm, table_hbm, ubias_hbm, ibias_hbm,
                  fcw_hbm, fcb_hbm, out_hbm,
                  uidx_v, iidx_v, iidx_off_v, urows_v, irows_v,
                  ubias_v, ibias_v, w_v, fcb_v, out_v,
                  sem_u, sem_i, sem_bu, sem_bi):
        wid = lax.axis_index("s") * num_cores + lax.axis_index("c")
        base = wid * bpw
        pltpu.sync_copy(uidx_hbm.at[pl.ds(base, bpw)], uidx_v)
        pltpu.sync_copy(iidx_hbm.at[pl.ds(base, bpw)], iidx_v)
        pltpu.sync_copy(fcw_hbm, w_v)
        pltpu.sync_copy(fcb_hbm, fcb_v)
        for j in range(bpw // _L):
            iidx_off_v[pl.ds(j * _L, _L)] = iidx_v[pl.ds(j * _L, _L)] + _F0
        cu = pltpu.async_copy(table_hbm.at[uidx_v], urows_v, sem_u)
        ci = pltpu.async_copy(table_hbm.at[iidx_off_v], irows_v, sem_i)
        cbu = pltpu.async_copy(ubias_hbm.at[uidx_v], ubias_v, sem_bu)
        cbi = pltpu.async_copy(ibias_hbm.at[iidx_v], ibias_v, sem_bi)
        cu.wait()
        ci.wait()
        cbu.wait()
        cbi.wait()
        w0v = w_v[pl.ds(0, _L)]
        w1v = w_v[pl.ds(_L, _L)]
        fbv = fcb_v[...]
        m_last = lax.iota(jnp.int32, _L) == (_L - 1)

        def body(r, carry):
            u0 = urows_v[r, pl.ds(0, _L)]
            u1 = urows_v[r, pl.ds(_L, _L)]
            i0 = irows_v[r, pl.ds(0, _L)]
            i1 = irows_v[r, pl.ds(_L, _L)]
            h = u0 * i0 * w0v + u1 * i1 * w1v
            c = plsc.cumsum(h)
            plsc.store_compressed(out_v.at[pl.ds(r, _L)], c, mask=m_last)
            return carry

        lax.fori_loop(0, bpw, body, 0)

        def body2(g, carry):
            o = g * _L
            out_v[pl.ds(o, _L)] = (out_v[pl.ds(o, _L)] + fbv
                                   + ubias_v[pl.ds(o, _L)]
                                   + ibias_v[pl.ds(o, _L)])
            return carry

        lax.fori_loop(0, bpw // _L, body2, 0)
        pltpu.sync_copy(out_v.at[pl.ds(0, bpw)], out_hbm.at[pl.ds(base, bpw)])

    return sc_kernel


def kernel(x, table, user_bias, item_bias, fc_w, fc_b):
    uidx = x[:, 0].astype(jnp.int32)
    iidx = x[:, 1].astype(jnp.int32)
    info = plsc.get_sparse_core_info()
    k = _make_sc_kernel(info.num_cores, info.num_subcores)
    return k(uidx, iidx, table, user_bias.reshape(-1), item_bias.reshape(-1),
             fc_w.reshape(-1), jnp.broadcast_to(fc_b.reshape(()), (_L,)))
